# Initial kernel scaffold; baseline (speedup 1.0000x reference)
import jax
import jax.numpy as jnp
from jax.experimental import pallas as pl


def kernel(hidden_states, gate_w, e_score_correction_bias, w1, w2, w3):
    raise NotImplementedError("write your pallas kernel here")


# speedup vs baseline: 1.1779x; 1.1779x over previous
import functools

import jax
import jax.numpy as jnp
from jax.experimental import pallas as pl
from jax.experimental.pallas import tpu as pltpu


def _moe_dense_body(x_ref, gw_ref, bias_ref, w1_ref, w2_ref, w3_ref,
                    out_ref, wfull_ref, *, n_chunks, chunk, n_experts):
    e = pl.program_id(0)
    f = pl.program_id(1)

    @pl.when((e == 0) & (f == 0))
    def _router():
        for c in range(n_chunks):
            sl = pl.ds(c * chunk, chunk)
            xc = x_ref[sl, :]
            logits = jax.lax.dot_general(
                xc, gw_ref[...], (((1,), (1,)), ((), ())),
                preferred_element_type=jnp.float32)
            rw = jax.nn.sigmoid(logits)
            scores = rw + bias_ref[...]
            colid = jax.lax.broadcasted_iota(jnp.int32, (chunk, n_experts), 1)
            v1 = jnp.max(scores, axis=1, keepdims=True)
            i1 = jnp.min(jnp.where(scores == v1, colid, n_experts),
                         axis=1, keepdims=True)
            m1 = colid == i1
            s2 = jnp.where(m1, -jnp.inf, scores)
            v2 = jnp.max(s2, axis=1, keepdims=True)
            i2 = jnp.min(jnp.where(s2 == v2, colid, n_experts),
                         axis=1, keepdims=True)
            m2 = colid == i2
            wv1 = jnp.sum(jnp.where(m1, rw, 0.0), axis=1, keepdims=True)
            wv2 = jnp.sum(jnp.where(m2, rw, 0.0), axis=1, keepdims=True)
            denom = wv1 + wv2
            wfull = (jnp.where(m1, wv1, 0.0) + jnp.where(m2, wv2, 0.0)) / denom
            wfull_ref[sl, :] = wfull

    w1b = w1_ref[0].astype(jnp.bfloat16)
    w3b = w3_ref[0].astype(jnp.bfloat16)
    w2b = w2_ref[0].astype(jnp.bfloat16)
    eoh = jax.lax.broadcasted_iota(jnp.int32, (1, n_experts), 1) == e
    first = (e == 0) & (f == 0)
    for c in range(n_chunks):
        sl = pl.ds(c * chunk, chunk)
        xb = x_ref[sl, :].astype(jnp.bfloat16)
        a = jax.lax.dot_general(xb, w1b, (((1,), (1,)), ((), ())),
                                preferred_element_type=jnp.float32)
        b = jax.lax.dot_general(xb, w3b, (((1,), (1,)), ((), ())),
                                preferred_element_type=jnp.float32)
        h = (a * jax.nn.sigmoid(a)) * b
        o = jax.lax.dot_general(h.astype(jnp.bfloat16), w2b,
                                (((1,), (1,)), ((), ())),
                                preferred_element_type=jnp.float32)
        we = jnp.sum(jnp.where(eoh, wfull_ref[sl, :], 0.0),
                     axis=1, keepdims=True)
        o = o * we

        @pl.when(first)
        def _init():
            out_ref[sl, :] = o

        @pl.when(jnp.logical_not(first))
        def _acc():
            out_ref[sl, :] = out_ref[sl, :] + o


def kernel(hidden_states, gate_w, e_score_correction_bias, w1, w2, w3):
    bsz, seq, d_model = hidden_states.shape
    n_experts, d_ff, _ = w1.shape
    tokens = bsz * seq
    x = hidden_states.reshape(tokens, d_model)
    bias2d = e_score_correction_bias.reshape(1, n_experts)

    chunk = 256 if tokens % 256 == 0 else tokens
    n_chunks = tokens // chunk
    fblk = 256 if d_ff % 256 == 0 else d_ff
    n_fblk = d_ff // fblk

    body = functools.partial(_moe_dense_body, n_chunks=n_chunks, chunk=chunk,
                             n_experts=n_experts)

    out = pl.pallas_call(
        body,
        grid=(n_experts, n_fblk),
        in_specs=[
            pl.BlockSpec((tokens, d_model), lambda e, f: (0, 0)),
            pl.BlockSpec((n_experts, d_model), lambda e, f: (0, 0)),
            pl.BlockSpec((1, n_experts), lambda e, f: (0, 0)),
            pl.BlockSpec((1, fblk, d_model), lambda e, f: (e, f, 0)),
            pl.BlockSpec((1, d_model, fblk), lambda e, f: (e, 0, f)),
            pl.BlockSpec((1, fblk, d_model), lambda e, f: (e, f, 0)),
        ],
        out_specs=pl.BlockSpec((tokens, d_model), lambda e, f: (0, 0)),
        out_shape=jax.ShapeDtypeStruct((tokens, d_model), jnp.float32),
        scratch_shapes=[pltpu.VMEM((tokens, n_experts), jnp.float32)],
        compiler_params=pltpu.CompilerParams(
            dimension_semantics=("arbitrary", "arbitrary")),
    )(x, gate_w, bias2d, w1, w2, w3)
    return out.reshape(bsz, seq, d_model)


# speedup vs baseline: 1.1985x; 1.0175x over previous
import functools

import jax
import jax.numpy as jnp
from jax.experimental import pallas as pl
from jax.experimental.pallas import tpu as pltpu

_BM = 128


def _router_body(x_ref, gw_ref, bias_ref, pos_ref, wout_ref, counts_ref,
                 m1_ref, m2_ref, *, n_chunks, chunk, n_experts, bm):
    counts = jnp.zeros((1, n_experts), jnp.float32)
    for c in range(n_chunks):
        sl = pl.ds(c * chunk, chunk)
        xc = x_ref[sl, :]
        logits = jax.lax.dot_general(
            xc, gw_ref[...], (((1,), (1,)), ((), ())),
            preferred_element_type=jnp.float32)
        rw = jax.nn.sigmoid(logits)
        scores = rw + bias_ref[...]
        colid = jax.lax.broadcasted_iota(jnp.int32, (chunk, n_experts), 1)
        v1 = jnp.max(scores, axis=1, keepdims=True)
        i1 = jnp.min(jnp.where(scores == v1, colid, n_experts),
                     axis=1, keepdims=True)
        m1 = (colid == i1).astype(jnp.float32)
        s2 = jnp.where(m1 > 0, -jnp.inf, scores)
        v2 = jnp.max(s2, axis=1, keepdims=True)
        i2 = jnp.min(jnp.where(s2 == v2, colid, n_experts),
                     axis=1, keepdims=True)
        m2 = (colid == i2).astype(jnp.float32)
        wv1 = jnp.sum(m1 * rw, axis=1, keepdims=True)
        wv2 = jnp.sum(m2 * rw, axis=1, keepdims=True)
        denom = wv1 + wv2
        wout_ref[sl, :] = jnp.concatenate([wv1 / denom, wv2 / denom], axis=1)
        m1_ref[sl, :] = m1
        m2_ref[sl, :] = m2
        counts = counts + jnp.sum(m1 + m2, axis=0, keepdims=True)

    counts_ref[...] = counts.astype(jnp.int32)
    ntiles = jnp.floor((counts + jnp.float32(bm - 1)) * (1.0 / bm))
    r8 = jax.lax.broadcasted_iota(jnp.int32, (n_experts, n_experts), 0)
    c8 = jax.lax.broadcasted_iota(jnp.int32, (n_experts, n_experts), 1)
    upper = (r8 < c8).astype(jnp.float32)
    padded_off = jnp.float32(bm) * jax.lax.dot_general(
        ntiles, upper, (((1,), (0,)), ((), ())),
        preferred_element_type=jnp.float32)

    rr = jax.lax.broadcasted_iota(jnp.int32, (chunk, chunk), 0)
    cc = jax.lax.broadcasted_iota(jnp.int32, (chunk, chunk), 1)
    tril = (rr > cc).astype(jnp.float32)
    base = jnp.zeros((1, n_experts), jnp.float32)
    for c in range(n_chunks):
        sl = pl.ds(c * chunk, chunk)
        m1 = m1_ref[sl, :]
        m2 = m2_ref[sl, :]
        m12 = m1 + m2
        cum = jax.lax.dot_general(tril, m12, (((1,), (0,)), ((), ())),
                                  preferred_element_type=jnp.float32) + base
        slot = cum + padded_off
        p0 = jnp.sum(m1 * slot, axis=1, keepdims=True)
        p1 = jnp.sum(m2 * slot, axis=1, keepdims=True)
        pos_ref[sl, :] = jnp.concatenate([p0, p1], axis=1).astype(jnp.int32)
        base = base + jnp.sum(m12, axis=0, keepdims=True)


def _router(x, gate_w, bias2d, n_experts, tokens, d_model, bm):
    chunk = 256 if tokens % 256 == 0 else tokens
    n_chunks = tokens // chunk
    body = functools.partial(_router_body, n_chunks=n_chunks, chunk=chunk,
                             n_experts=n_experts, bm=bm)
    return pl.pallas_call(
        body,
        grid=(1,),
        in_specs=[
            pl.BlockSpec((tokens, d_model), lambda i: (0, 0)),
            pl.BlockSpec((n_experts, d_model), lambda i: (0, 0)),
            pl.BlockSpec((1, n_experts), lambda i: (0, 0)),
        ],
        out_specs=[
            pl.BlockSpec((tokens, 2), lambda i: (0, 0)),
            pl.BlockSpec((tokens, 2), lambda i: (0, 0)),
            pl.BlockSpec((1, n_experts), lambda i: (0, 0)),
        ],
        out_shape=[
            jax.ShapeDtypeStruct((tokens, 2), jnp.int32),
            jax.ShapeDtypeStruct((tokens, 2), jnp.float32),
            jax.ShapeDtypeStruct((1, n_experts), jnp.int32),
        ],
        scratch_shapes=[pltpu.VMEM((tokens, n_experts), jnp.float32),
                        pltpu.VMEM((tokens, n_experts), jnp.float32)],
        compiler_params=pltpu.CompilerParams(
            dimension_semantics=("arbitrary",)),
    )(x, gate_w, bias2d)


def _gmm_body(eid_ref, xs_ref, w1_ref, w2_ref, w3_ref, out_ref, *, d_ff):
    fb = d_ff // 4 if d_ff % 4 == 0 else d_ff
    xb = xs_ref[...].astype(jnp.bfloat16)
    o = None
    for fi in range(d_ff // fb):
        fsl = pl.ds(fi * fb, fb)
        w1b = w1_ref[0, fsl, :].astype(jnp.bfloat16)
        w3b = w3_ref[0, fsl, :].astype(jnp.bfloat16)
        a = jax.lax.dot_general(xb, w1b, (((1,), (1,)), ((), ())),
                                preferred_element_type=jnp.float32)
        b = jax.lax.dot_general(xb, w3b, (((1,), (1,)), ((), ())),
                                preferred_element_type=jnp.float32)
        h = (a * jax.nn.sigmoid(a)) * b
        w2b = w2_ref[0, :, fsl].astype(jnp.bfloat16)
        op = jax.lax.dot_general(h.astype(jnp.bfloat16), w2b,
                                 (((1,), (1,)), ((), ())),
                                 preferred_element_type=jnp.float32)
        o = op if o is None else o + op
    out_ref[...] = o


def _gmm(step_eid, x_sorted, w1, w2, w3, n_blocks, d_model, d_ff):
    body = functools.partial(_gmm_body, d_ff=d_ff)
    grid_spec = pltpu.PrefetchScalarGridSpec(
        num_scalar_prefetch=1,
        grid=(n_blocks,),
        in_specs=[
            pl.BlockSpec((_BM, d_model), lambda s, eid: (s, 0)),
            pl.BlockSpec((1, d_ff, d_model), lambda s, eid: (eid[s], 0, 0)),
            pl.BlockSpec((1, d_model, d_ff), lambda s, eid: (eid[s], 0, 0)),
            pl.BlockSpec((1, d_ff, d_model), lambda s, eid: (eid[s], 0, 0)),
        ],
        out_specs=pl.BlockSpec((_BM, d_model), lambda s, eid: (s, 0)),
    )
    return pl.pallas_call(
        body,
        grid_spec=grid_spec,
        out_shape=jax.ShapeDtypeStruct((n_blocks * _BM, d_model),
                                       jnp.float32),
        compiler_params=pltpu.CompilerParams(
            dimension_semantics=("arbitrary",)),
    )(step_eid, x_sorted, w1, w2, w3)


def kernel(hidden_states, gate_w, e_score_correction_bias, w1, w2, w3):
    bsz, seq, d_model = hidden_states.shape
    n_experts, d_ff, _ = w1.shape
    tokens = bsz * seq
    n_pairs = 2 * tokens
    n_blocks = n_pairs // _BM + n_experts
    pad_rows = n_blocks * _BM
    x = hidden_states.reshape(tokens, d_model)
    bias2d = e_score_correction_bias.reshape(1, n_experts)

    pos, wout, counts = _router(x, gate_w, bias2d, n_experts, tokens,
                                d_model, _BM)

    ntiles = (counts[0] + (_BM - 1)) // _BM
    starts = jnp.concatenate(
        [jnp.zeros((1,), jnp.int32), jnp.cumsum(ntiles)[:-1]])
    step_eid = (jnp.sum(
        (jnp.arange(n_blocks, dtype=jnp.int32)[:, None]
         >= starts[None, :]).astype(jnp.int32), axis=1) - 1).astype(jnp.int32)
    step_eid = jnp.clip(step_eid, 0, n_experts - 1)

    tid_src = jnp.repeat(jnp.arange(tokens, dtype=jnp.int32), 2)
    sorted_tid = jnp.zeros((pad_rows,), jnp.int32).at[
        pos.reshape(-1)].set(tid_src, mode="drop")
    x_sorted = x[jnp.clip(sorted_tid, 0, tokens - 1)]

    out_sorted = _gmm(step_eid, x_sorted, w1, w2, w3, n_blocks, d_model,
                      d_ff)

    final = (out_sorted[pos[:, 0]] * wout[:, 0:1]
             + out_sorted[pos[:, 1]] * wout[:, 1:2])
    return final.reshape(bsz, seq, d_model)
